# Initial kernel scaffold; baseline (speedup 1.0000x reference)
import jax
import jax.numpy as jnp
from jax.experimental import pallas as pl


def kernel(x, edge_index, batch, W1, b1, W2, b2):
    raise NotImplementedError("write your pallas kernel here")


# speedup vs baseline: 73.3596x; 73.3596x over previous
import jax
import jax.numpy as jnp
from jax import lax
from jax.experimental import pallas as pl
from jax.experimental.pallas import tpu as pltpu
from jax.experimental.pallas import tpu_sc as plsc

NN = 100000
NE = 3200000
NG = 64
NPAD = 100096
NROWS = NPAD // 128
SUB = 16
CORES = 2
NW = CORES * SUB
PER_SUB = NPAD // SUB

RPW = 784
EP_ROWS = RPW * NW
EP = EP_ROWS * 128
U = 8
T_OUT = RPW // U

_mesh = plsc.VectorSubcoreMesh(core_axis_name="c", subcore_axis_name="s")


def _deg_body(dst_hbm, ones_hbm, zeros_hbm, out_hbm, deg_sh, idx_v, ones_v, stage_v):
    c = lax.axis_index("c")
    s = lax.axis_index("s")
    wid = c * SUB + s
    sl = pl.ds(s * PER_SUB, PER_SUB)
    pltpu.sync_copy(zeros_hbm.at[sl], stage_v)
    pltpu.sync_copy(stage_v, deg_sh.at[sl])
    pltpu.sync_copy(ones_hbm, ones_v)
    plsc.subcore_barrier()
    base = wid * RPW

    def outer(i, carry):
        r0 = base + i * U
        pltpu.sync_copy(dst_hbm.at[pl.ds(r0, U), :], idx_v)
        for j in range(U):
            pltpu.sync_copy(ones_v, deg_sh.at[idx_v.at[j]], add=True)
        return carry

    lax.fori_loop(0, T_OUT, outer, 0)
    plsc.subcore_barrier()
    pltpu.sync_copy(deg_sh.at[sl], stage_v)
    pltpu.sync_copy(stage_v, out_hbm.at[pl.ds(c * NPAD + s * PER_SUB, PER_SUB)])


_deg_call = pl.kernel(
    _deg_body,
    out_type=jax.ShapeDtypeStruct((CORES * NPAD,), jnp.float32),
    mesh=_mesh,
    scratch_types=[
        pltpu.VMEM_SHARED((NPAD,), jnp.float32),
        pltpu.VMEM((U, 128), jnp.int32),
        pltpu.VMEM((128,), jnp.float32),
        pltpu.VMEM((PER_SUB,), jnp.float32),
    ],
)


def _p1_body(src_hbm, dst_hbm, v1_hbm, zeros_hbm, out_hbm,
             v1_sh, acc_sh, idx_s, idx_d, vals, stage_v):
    c = lax.axis_index("c")
    s = lax.axis_index("s")
    wid = c * SUB + s
    sl = pl.ds(s * PER_SUB, PER_SUB)
    pltpu.sync_copy(v1_hbm.at[sl], stage_v)
    pltpu.sync_copy(stage_v, v1_sh.at[sl])
    pltpu.sync_copy(zeros_hbm.at[sl], stage_v)
    pltpu.sync_copy(stage_v, acc_sh.at[sl])
    plsc.subcore_barrier()
    base = wid * RPW

    def outer(i, carry):
        r0 = base + i * U
        pltpu.sync_copy(src_hbm.at[pl.ds(r0, U), :], idx_s)
        pltpu.sync_copy(dst_hbm.at[pl.ds(r0, U), :], idx_d)
        for j in range(U):
            pltpu.sync_copy(v1_sh.at[idx_s.at[j]], vals.at[j])
            pltpu.sync_copy(vals.at[j], acc_sh.at[idx_d.at[j]], add=True)
        return carry

    lax.fori_loop(0, T_OUT, outer, 0)
    plsc.subcore_barrier()
    pltpu.sync_copy(acc_sh.at[sl], stage_v)
    pltpu.sync_copy(stage_v, out_hbm.at[pl.ds(c * NPAD + s * PER_SUB, PER_SUB)])


_p1_call = pl.kernel(
    _p1_body,
    out_type=jax.ShapeDtypeStruct((CORES * NPAD,), jnp.float32),
    mesh=_mesh,
    scratch_types=[
        pltpu.VMEM_SHARED((NPAD,), jnp.float32),
        pltpu.VMEM_SHARED((NPAD,), jnp.float32),
        pltpu.VMEM((U, 128), jnp.int32),
        pltpu.VMEM((U, 128), jnp.int32),
        pltpu.VMEM((U, 128), jnp.float32),
        pltpu.VMEM((PER_SUB,), jnp.float32),
    ],
)


U2 = 4
T_OUT2 = RPW // U2


def _p2_body(src_hbm, dst_hbm, v2c0_hbm, v2c1_hbm, zeros_hbm, out0_hbm, out1_hbm,
             v0_sh, v1_sh, a0_sh, a1_sh, idx_s, idx_d, vals0, vals1, stage_v):
    c = lax.axis_index("c")
    s = lax.axis_index("s")
    wid = c * SUB + s
    sl = pl.ds(s * PER_SUB, PER_SUB)
    pltpu.sync_copy(v2c0_hbm.at[sl], stage_v)
    pltpu.sync_copy(stage_v, v0_sh.at[sl])
    pltpu.sync_copy(v2c1_hbm.at[sl], stage_v)
    pltpu.sync_copy(stage_v, v1_sh.at[sl])
    pltpu.sync_copy(zeros_hbm.at[sl], stage_v)
    pltpu.sync_copy(stage_v, a0_sh.at[sl])
    pltpu.sync_copy(stage_v, a1_sh.at[sl])
    plsc.subcore_barrier()
    base = wid * RPW

    def outer(i, carry):
        r0 = base + i * U2
        pltpu.sync_copy(src_hbm.at[pl.ds(r0, U2), :], idx_s)
        pltpu.sync_copy(dst_hbm.at[pl.ds(r0, U2), :], idx_d)
        for j in range(U2):
            pltpu.sync_copy(v0_sh.at[idx_s.at[j]], vals0.at[j])
            pltpu.sync_copy(v1_sh.at[idx_s.at[j]], vals1.at[j])
            pltpu.sync_copy(vals0.at[j], a0_sh.at[idx_d.at[j]], add=True)
            pltpu.sync_copy(vals1.at[j], a1_sh.at[idx_d.at[j]], add=True)
        return carry

    lax.fori_loop(0, T_OUT2, outer, 0)
    plsc.subcore_barrier()
    pltpu.sync_copy(a0_sh.at[sl], stage_v)
    pltpu.sync_copy(stage_v, out0_hbm.at[pl.ds(c * NPAD + s * PER_SUB, PER_SUB)])
    pltpu.sync_copy(a1_sh.at[sl], stage_v)
    pltpu.sync_copy(stage_v, out1_hbm.at[pl.ds(c * NPAD + s * PER_SUB, PER_SUB)])


_p2_call = pl.kernel(
    _p2_body,
    out_type=(
        jax.ShapeDtypeStruct((CORES * NPAD,), jnp.float32),
        jax.ShapeDtypeStruct((CORES * NPAD,), jnp.float32),
    ),
    mesh=_mesh,
    scratch_types=[
        pltpu.VMEM_SHARED((NPAD,), jnp.float32),
        pltpu.VMEM_SHARED((NPAD,), jnp.float32),
        pltpu.VMEM_SHARED((NPAD,), jnp.float32),
        pltpu.VMEM_SHARED((NPAD,), jnp.float32),
        pltpu.VMEM((U2, 128), jnp.int32),
        pltpu.VMEM((U2, 128), jnp.int32),
        pltpu.VMEM((U2, 128), jnp.float32),
        pltpu.VMEM((U2, 128), jnp.float32),
        pltpu.VMEM((PER_SUB,), jnp.float32),
    ],
)


def _prep_body(degp_ref, xp_ref, dinv_ref, v1_ref):
    deg = degp_ref[0] + degp_ref[1] + 1.0
    dinv = lax.rsqrt(deg)
    dinv_ref[...] = dinv
    v1_ref[...] = xp_ref[...] * dinv


_prep_call = pl.pallas_call(
    _prep_body,
    out_shape=(
        jax.ShapeDtypeStruct((NROWS, 128), jnp.float32),
        jax.ShapeDtypeStruct((NROWS, 128), jnp.float32),
    ),
)


def _mid_body(accp_ref, dinv_ref, v1_ref, w1_ref, b1_ref, w2_ref, v2cm_ref):
    dinv = dinv_ref[...]
    s1 = dinv * (accp_ref[0] + accp_ref[1] + v1_ref[...])
    g0 = jnp.zeros_like(s1)
    g1 = jnp.zeros_like(s1)
    for k in range(16):
        h = jnp.maximum(s1 * w1_ref[0, k] + b1_ref[0, k], 0.0)
        g0 += h * w2_ref[k, 0]
        g1 += h * w2_ref[k, 1]
    v2cm_ref[0] = g0 * dinv
    v2cm_ref[1] = g1 * dinv


_mid_call = pl.pallas_call(
    _mid_body,
    in_specs=[
        pl.BlockSpec(memory_space=pltpu.VMEM),
        pl.BlockSpec(memory_space=pltpu.VMEM),
        pl.BlockSpec(memory_space=pltpu.VMEM),
        pl.BlockSpec(memory_space=pltpu.SMEM),
        pl.BlockSpec(memory_space=pltpu.SMEM),
        pl.BlockSpec(memory_space=pltpu.SMEM),
    ],
    out_shape=jax.ShapeDtypeStruct((2, NROWS, 128), jnp.float32),
)


def _final_body(acc2cm_ref, v2cm_ref, dinv_ref, b2_ref, batch_ref, out_ref):
    dinv = dinv_ref[...]
    hs = []
    for ch in range(2):
        s2 = dinv * (acc2cm_ref[0, ch] + acc2cm_ref[1, ch] + v2cm_ref[ch])
        hs.append(jnp.maximum(s2 + b2_ref[0, ch], 0.0))
    bt = batch_ref[...]
    for g in range(NG):
        m = (bt == g).astype(jnp.float32)
        cnt = jnp.maximum(jnp.sum(m), 1.0)
        z0 = jnp.sum(m * hs[0]) / cnt
        z1 = jnp.sum(m * hs[1]) / cnt
        mx = jnp.maximum(z0, z1)
        lse = jnp.log(jnp.exp(z0 - mx) + jnp.exp(z1 - mx)) + mx
        out_ref[g, 0] = z0 - lse
        out_ref[g, 1] = z1 - lse


_final_call = pl.pallas_call(
    _final_body,
    in_specs=[
        pl.BlockSpec(memory_space=pltpu.VMEM),
        pl.BlockSpec(memory_space=pltpu.VMEM),
        pl.BlockSpec(memory_space=pltpu.VMEM),
        pl.BlockSpec(memory_space=pltpu.SMEM),
        pl.BlockSpec(memory_space=pltpu.VMEM),
    ],
    out_specs=pl.BlockSpec(memory_space=pltpu.SMEM),
    out_shape=jax.ShapeDtypeStruct((NG, 2), jnp.float32),
)


def kernel(x, edge_index, batch, W1, b1, W2, b2):
    ei = edge_index.astype(jnp.int32)
    npad_e = EP - NE
    pad_idx = NN + (jnp.arange(npad_e, dtype=jnp.int32) % (NPAD - NN))
    srcp = jnp.concatenate([ei[0], pad_idx]).reshape(EP_ROWS, 128)
    dstp = jnp.concatenate([ei[1], pad_idx]).reshape(EP_ROWS, 128)
    zerosN = jnp.zeros((NPAD,), jnp.float32)
    ones128 = jnp.ones((128,), jnp.float32)

    degp = _deg_call(dstp, ones128, zerosN)

    xp = jnp.concatenate([x[:, 0], jnp.zeros((NPAD - NN,), jnp.float32)])
    dinv, v1 = _prep_call(degp.reshape(CORES, NROWS, 128), xp.reshape(NROWS, 128))

    acc1p = _p1_call(srcp, dstp, v1.reshape(NPAD), zerosN)

    v2cm = _mid_call(acc1p.reshape(CORES, NROWS, 128), dinv, v1,
                     W1, b1.reshape(1, 16), W2)

    v2flat = v2cm.reshape(2, NPAD)
    acc2c0, acc2c1 = _p2_call(srcp, dstp, v2flat[0], v2flat[1], zerosN)

    acc2cm = jnp.stack(
        [acc2c0.reshape(CORES, NROWS, 128), acc2c1.reshape(CORES, NROWS, 128)],
        axis=1)
    batchp = jnp.concatenate(
        [batch.astype(jnp.int32), jnp.full((NPAD - NN,), NG, jnp.int32)]
    ).reshape(NROWS, 128)

    return _final_call(acc2cm, v2cm, dinv, b2.reshape(1, 2), batchp)


# speedup vs baseline: 121.3604x; 1.6543x over previous
import jax
import jax.numpy as jnp
from jax import lax
from jax.experimental import pallas as pl
from jax.experimental.pallas import tpu as pltpu
from jax.experimental.pallas import tpu_sc as plsc

NN = 100000
NE = 3200000
NG = 64
NPAD = 100096
NROWS = NPAD // 128
SUB = 16
CORES = 2
NW = CORES * SUB
PER_SUB = NPAD // SUB

RPW = 784
EP_ROWS = RPW * NW
EP = EP_ROWS * 128
U = 8
T_OUT = RPW // U

_mesh = plsc.VectorSubcoreMesh(core_axis_name="c", subcore_axis_name="s")


def _deg_body(dst_hbm, ones_hbm, zeros_hbm, out_hbm, deg_sh, idx_v, ones_v, stage_v):
    c = lax.axis_index("c")
    s = lax.axis_index("s")
    wid = c * SUB + s
    sl = pl.ds(s * PER_SUB, PER_SUB)
    pltpu.sync_copy(zeros_hbm.at[sl], stage_v)
    pltpu.sync_copy(stage_v, deg_sh.at[sl])
    pltpu.sync_copy(ones_hbm.at[pl.ds(0, U * 128)], ones_v)
    plsc.subcore_barrier()
    base = wid * RPW

    def outer(i, carry):
        e0 = base * 128 + i * (U * 128)
        pltpu.sync_copy(dst_hbm.at[pl.ds(e0, U * 128)], idx_v)
        pltpu.sync_copy(ones_v, deg_sh.at[idx_v], add=True)
        return carry

    lax.fori_loop(0, T_OUT, outer, 0)
    plsc.subcore_barrier()
    pltpu.sync_copy(deg_sh.at[sl], stage_v)
    pltpu.sync_copy(stage_v, out_hbm.at[pl.ds(c * NPAD + s * PER_SUB, PER_SUB)])


_deg_call = pl.kernel(
    _deg_body,
    out_type=jax.ShapeDtypeStruct((CORES * NPAD,), jnp.float32),
    mesh=_mesh,
    scratch_types=[
        pltpu.VMEM_SHARED((NPAD,), jnp.float32),
        pltpu.VMEM((U * 128,), jnp.int32),
        pltpu.VMEM((U * 128,), jnp.float32),
        pltpu.VMEM((PER_SUB,), jnp.float32),
    ],
)


def _p1_body(src_hbm, dst_hbm, v1_hbm, zeros_hbm, out_hbm,
             v1_sh, acc_sh, idx_s, idx_d, vals, stage_v):
    c = lax.axis_index("c")
    s = lax.axis_index("s")
    wid = c * SUB + s
    sl = pl.ds(s * PER_SUB, PER_SUB)
    pltpu.sync_copy(v1_hbm.at[sl], stage_v)
    pltpu.sync_copy(stage_v, v1_sh.at[sl])
    pltpu.sync_copy(zeros_hbm.at[sl], stage_v)
    pltpu.sync_copy(stage_v, acc_sh.at[sl])
    plsc.subcore_barrier()
    base = wid * RPW

    def outer(i, carry):
        e0 = base * 128 + i * (U * 128)
        pltpu.sync_copy(src_hbm.at[pl.ds(e0, U * 128)], idx_s)
        pltpu.sync_copy(dst_hbm.at[pl.ds(e0, U * 128)], idx_d)
        pltpu.sync_copy(v1_sh.at[idx_s], vals)
        pltpu.sync_copy(vals, acc_sh.at[idx_d], add=True)
        return carry

    lax.fori_loop(0, T_OUT, outer, 0)
    plsc.subcore_barrier()
    pltpu.sync_copy(acc_sh.at[sl], stage_v)
    pltpu.sync_copy(stage_v, out_hbm.at[pl.ds(c * NPAD + s * PER_SUB, PER_SUB)])


_p1_call = pl.kernel(
    _p1_body,
    out_type=jax.ShapeDtypeStruct((CORES * NPAD,), jnp.float32),
    mesh=_mesh,
    scratch_types=[
        pltpu.VMEM_SHARED((NPAD,), jnp.float32),
        pltpu.VMEM_SHARED((NPAD,), jnp.float32),
        pltpu.VMEM((U * 128,), jnp.int32),
        pltpu.VMEM((U * 128,), jnp.int32),
        pltpu.VMEM((U * 128,), jnp.float32),
        pltpu.VMEM((PER_SUB,), jnp.float32),
    ],
)


U2 = 8
T_OUT2 = RPW // U2


def _p2_body(src_hbm, dst_hbm, v2c0_hbm, v2c1_hbm, zeros_hbm, out0_hbm, out1_hbm,
             v0_sh, v1_sh, a0_sh, a1_sh, idx_s, idx_d, vals0, vals1, stage_v):
    c = lax.axis_index("c")
    s = lax.axis_index("s")
    wid = c * SUB + s
    sl = pl.ds(s * PER_SUB, PER_SUB)
    pltpu.sync_copy(v2c0_hbm.at[sl], stage_v)
    pltpu.sync_copy(stage_v, v0_sh.at[sl])
    pltpu.sync_copy(v2c1_hbm.at[sl], stage_v)
    pltpu.sync_copy(stage_v, v1_sh.at[sl])
    pltpu.sync_copy(zeros_hbm.at[sl], stage_v)
    pltpu.sync_copy(stage_v, a0_sh.at[sl])
    pltpu.sync_copy(stage_v, a1_sh.at[sl])
    plsc.subcore_barrier()
    base = wid * RPW

    def outer(i, carry):
        e0 = base * 128 + i * (U2 * 128)
        pltpu.sync_copy(src_hbm.at[pl.ds(e0, U2 * 128)], idx_s)
        pltpu.sync_copy(dst_hbm.at[pl.ds(e0, U2 * 128)], idx_d)
        pltpu.sync_copy(v0_sh.at[idx_s], vals0)
        pltpu.sync_copy(v1_sh.at[idx_s], vals1)
        pltpu.sync_copy(vals0, a0_sh.at[idx_d], add=True)
        pltpu.sync_copy(vals1, a1_sh.at[idx_d], add=True)
        return carry

    lax.fori_loop(0, T_OUT2, outer, 0)
    plsc.subcore_barrier()
    pltpu.sync_copy(a0_sh.at[sl], stage_v)
    pltpu.sync_copy(stage_v, out0_hbm.at[pl.ds(c * NPAD + s * PER_SUB, PER_SUB)])
    pltpu.sync_copy(a1_sh.at[sl], stage_v)
    pltpu.sync_copy(stage_v, out1_hbm.at[pl.ds(c * NPAD + s * PER_SUB, PER_SUB)])


_p2_call = pl.kernel(
    _p2_body,
    out_type=(
        jax.ShapeDtypeStruct((CORES * NPAD,), jnp.float32),
        jax.ShapeDtypeStruct((CORES * NPAD,), jnp.float32),
    ),
    mesh=_mesh,
    scratch_types=[
        pltpu.VMEM_SHARED((NPAD,), jnp.float32),
        pltpu.VMEM_SHARED((NPAD,), jnp.float32),
        pltpu.VMEM_SHARED((NPAD,), jnp.float32),
        pltpu.VMEM_SHARED((NPAD,), jnp.float32),
        pltpu.VMEM((U2 * 128,), jnp.int32),
        pltpu.VMEM((U2 * 128,), jnp.int32),
        pltpu.VMEM((U2 * 128,), jnp.float32),
        pltpu.VMEM((U2 * 128,), jnp.float32),
        pltpu.VMEM((PER_SUB,), jnp.float32),
    ],
)


def _prep_body(degp_ref, xp_ref, dinv_ref, v1_ref):
    deg = degp_ref[0] + degp_ref[1] + 1.0
    dinv = lax.rsqrt(deg)
    dinv_ref[...] = dinv
    v1_ref[...] = xp_ref[...] * dinv


_prep_call = pl.pallas_call(
    _prep_body,
    out_shape=(
        jax.ShapeDtypeStruct((NROWS, 128), jnp.float32),
        jax.ShapeDtypeStruct((NROWS, 128), jnp.float32),
    ),
)


def _mid_body(accp_ref, dinv_ref, v1_ref, w1_ref, b1_ref, w2_ref, v2cm_ref):
    dinv = dinv_ref[...]
    s1 = dinv * (accp_ref[0] + accp_ref[1] + v1_ref[...])
    g0 = jnp.zeros_like(s1)
    g1 = jnp.zeros_like(s1)
    for k in range(16):
        h = jnp.maximum(s1 * w1_ref[0, k] + b1_ref[0, k], 0.0)
        g0 += h * w2_ref[k, 0]
        g1 += h * w2_ref[k, 1]
    v2cm_ref[0] = g0 * dinv
    v2cm_ref[1] = g1 * dinv


_mid_call = pl.pallas_call(
    _mid_body,
    in_specs=[
        pl.BlockSpec(memory_space=pltpu.VMEM),
        pl.BlockSpec(memory_space=pltpu.VMEM),
        pl.BlockSpec(memory_space=pltpu.VMEM),
        pl.BlockSpec(memory_space=pltpu.SMEM),
        pl.BlockSpec(memory_space=pltpu.SMEM),
        pl.BlockSpec(memory_space=pltpu.SMEM),
    ],
    out_shape=jax.ShapeDtypeStruct((2, NROWS, 128), jnp.float32),
)


def _final_body(acc2cm_ref, v2cm_ref, dinv_ref, b2_ref, batch_ref, out_ref):
    dinv = dinv_ref[...]
    hs = []
    for ch in range(2):
        s2 = dinv * (acc2cm_ref[0, ch] + acc2cm_ref[1, ch] + v2cm_ref[ch])
        hs.append(jnp.maximum(s2 + b2_ref[0, ch], 0.0))
    bt = batch_ref[...]
    for g in range(NG):
        m = (bt == g).astype(jnp.float32)
        cnt = jnp.maximum(jnp.sum(m), 1.0)
        z0 = jnp.sum(m * hs[0]) / cnt
        z1 = jnp.sum(m * hs[1]) / cnt
        mx = jnp.maximum(z0, z1)
        lse = jnp.log(jnp.exp(z0 - mx) + jnp.exp(z1 - mx)) + mx
        out_ref[g, 0] = z0 - lse
        out_ref[g, 1] = z1 - lse


_final_call = pl.pallas_call(
    _final_body,
    in_specs=[
        pl.BlockSpec(memory_space=pltpu.VMEM),
        pl.BlockSpec(memory_space=pltpu.VMEM),
        pl.BlockSpec(memory_space=pltpu.VMEM),
        pl.BlockSpec(memory_space=pltpu.SMEM),
        pl.BlockSpec(memory_space=pltpu.VMEM),
    ],
    out_specs=pl.BlockSpec(memory_space=pltpu.SMEM),
    out_shape=jax.ShapeDtypeStruct((NG, 2), jnp.float32),
)


def kernel(x, edge_index, batch, W1, b1, W2, b2):
    ei = edge_index.astype(jnp.int32)
    npad_e = EP - NE
    pad_idx = NN + (jnp.arange(npad_e, dtype=jnp.int32) % (NPAD - NN))
    srcp = jnp.concatenate([ei[0], pad_idx])
    dstp = jnp.concatenate([ei[1], pad_idx])
    zerosN = jnp.zeros((NPAD,), jnp.float32)
    ones128 = jnp.ones((U * 128,), jnp.float32)

    degp = _deg_call(dstp, ones128, zerosN)

    xp = jnp.concatenate([x[:, 0], jnp.zeros((NPAD - NN,), jnp.float32)])
    dinv, v1 = _prep_call(degp.reshape(CORES, NROWS, 128), xp.reshape(NROWS, 128))

    acc1p = _p1_call(srcp, dstp, v1.reshape(NPAD), zerosN)

    v2cm = _mid_call(acc1p.reshape(CORES, NROWS, 128), dinv, v1,
                     W1, b1.reshape(1, 16), W2)

    v2flat = v2cm.reshape(2, NPAD)
    acc2c0, acc2c1 = _p2_call(srcp, dstp, v2flat[0], v2flat[1], zerosN)

    acc2cm = jnp.stack(
        [acc2c0.reshape(CORES, NROWS, 128), acc2c1.reshape(CORES, NROWS, 128)],
        axis=1)
    batchp = jnp.concatenate(
        [batch.astype(jnp.int32), jnp.full((NPAD - NN,), NG, jnp.int32)]
    ).reshape(NROWS, 128)

    return _final_call(acc2cm, v2cm, dinv, b2.reshape(1, 2), batchp)


# speedup vs baseline: 160.8353x; 1.3253x over previous
import jax
import jax.numpy as jnp
from jax import lax
from jax.experimental import pallas as pl
from jax.experimental.pallas import tpu as pltpu
from jax.experimental.pallas import tpu_sc as plsc

NN = 100000
NE = 3200000
NG = 64
NPAD = 100096
NROWS = NPAD // 128
SUB = 16
CORES = 2
NW = CORES * SUB
PER_SUB = NPAD // SUB

RPW = 784
EP_ROWS = RPW * NW
EP = EP_ROWS * 128
U = 16
T_OUT = RPW // U

_mesh = plsc.VectorSubcoreMesh(core_axis_name="c", subcore_axis_name="s")


def _deg_body(dst_hbm, ones_hbm, zeros_hbm, out_hbm, deg_sh, idx_v, ones_v, stage_v):
    c = lax.axis_index("c")
    s = lax.axis_index("s")
    wid = c * SUB + s
    sl = pl.ds(s * PER_SUB, PER_SUB)
    pltpu.sync_copy(zeros_hbm.at[sl], stage_v)
    pltpu.sync_copy(stage_v, deg_sh.at[sl])
    pltpu.sync_copy(ones_hbm.at[pl.ds(0, U * 128)], ones_v)
    plsc.subcore_barrier()
    base = wid * RPW

    def outer(i, carry):
        e0 = base * 128 + i * (U * 128)
        pltpu.sync_copy(dst_hbm.at[pl.ds(e0, U * 128)], idx_v)
        pltpu.sync_copy(ones_v, deg_sh.at[idx_v], add=True)
        return carry

    lax.fori_loop(0, T_OUT, outer, 0)
    plsc.subcore_barrier()
    pltpu.sync_copy(deg_sh.at[sl], stage_v)
    pltpu.sync_copy(stage_v, out_hbm.at[pl.ds(c * NPAD + s * PER_SUB, PER_SUB)])


_deg_call = pl.kernel(
    _deg_body,
    out_type=jax.ShapeDtypeStruct((CORES * NPAD,), jnp.float32),
    mesh=_mesh,
    scratch_types=[
        pltpu.VMEM_SHARED((NPAD,), jnp.float32),
        pltpu.VMEM((U * 128,), jnp.int32),
        pltpu.VMEM((U * 128,), jnp.float32),
        pltpu.VMEM((PER_SUB,), jnp.float32),
    ],
)


def _p1_body(src_hbm, dst_hbm, v1_hbm, zeros_hbm, out_hbm,
             v1_sh, acc_sh, idx_s, idx_d, vals, stage_v):
    c = lax.axis_index("c")
    s = lax.axis_index("s")
    wid = c * SUB + s
    sl = pl.ds(s * PER_SUB, PER_SUB)
    pltpu.sync_copy(v1_hbm.at[sl], stage_v)
    pltpu.sync_copy(stage_v, v1_sh.at[sl])
    pltpu.sync_copy(zeros_hbm.at[sl], stage_v)
    pltpu.sync_copy(stage_v, acc_sh.at[sl])
    plsc.subcore_barrier()
    base = wid * RPW

    def outer(i, carry):
        e0 = base * 128 + i * (U * 128)
        pltpu.sync_copy(src_hbm.at[pl.ds(e0, U * 128)], idx_s)
        pltpu.sync_copy(dst_hbm.at[pl.ds(e0, U * 128)], idx_d)
        pltpu.sync_copy(v1_sh.at[idx_s], vals)
        pltpu.sync_copy(vals, acc_sh.at[idx_d], add=True)
        return carry

    lax.fori_loop(0, T_OUT, outer, 0)
    plsc.subcore_barrier()
    pltpu.sync_copy(acc_sh.at[sl], stage_v)
    pltpu.sync_copy(stage_v, out_hbm.at[pl.ds(c * NPAD + s * PER_SUB, PER_SUB)])


_p1_call = pl.kernel(
    _p1_body,
    out_type=jax.ShapeDtypeStruct((CORES * NPAD,), jnp.float32),
    mesh=_mesh,
    scratch_types=[
        pltpu.VMEM_SHARED((NPAD,), jnp.float32),
        pltpu.VMEM_SHARED((NPAD,), jnp.float32),
        pltpu.VMEM((U * 128,), jnp.int32),
        pltpu.VMEM((U * 128,), jnp.int32),
        pltpu.VMEM((U * 128,), jnp.float32),
        pltpu.VMEM((PER_SUB,), jnp.float32),
    ],
)


U2 = 16
T_OUT2 = RPW // U2


def _p2_body(src_hbm, dst_hbm, v2c0_hbm, v2c1_hbm, zeros_hbm, out0_hbm, out1_hbm,
             v0_sh, v1_sh, a0_sh, a1_sh, idx_s, idx_d, vals0, vals1, stage_v):
    c = lax.axis_index("c")
    s = lax.axis_index("s")
    wid = c * SUB + s
    sl = pl.ds(s * PER_SUB, PER_SUB)
    pltpu.sync_copy(v2c0_hbm.at[sl], stage_v)
    pltpu.sync_copy(stage_v, v0_sh.at[sl])
    pltpu.sync_copy(v2c1_hbm.at[sl], stage_v)
    pltpu.sync_copy(stage_v, v1_sh.at[sl])
    pltpu.sync_copy(zeros_hbm.at[sl], stage_v)
    pltpu.sync_copy(stage_v, a0_sh.at[sl])
    pltpu.sync_copy(stage_v, a1_sh.at[sl])
    plsc.subcore_barrier()
    base = wid * RPW

    def outer(i, carry):
        e0 = base * 128 + i * (U2 * 128)
        pltpu.sync_copy(src_hbm.at[pl.ds(e0, U2 * 128)], idx_s)
        pltpu.sync_copy(dst_hbm.at[pl.ds(e0, U2 * 128)], idx_d)
        pltpu.sync_copy(v0_sh.at[idx_s], vals0)
        pltpu.sync_copy(v1_sh.at[idx_s], vals1)
        pltpu.sync_copy(vals0, a0_sh.at[idx_d], add=True)
        pltpu.sync_copy(vals1, a1_sh.at[idx_d], add=True)
        return carry

    lax.fori_loop(0, T_OUT2, outer, 0)
    plsc.subcore_barrier()
    pltpu.sync_copy(a0_sh.at[sl], stage_v)
    pltpu.sync_copy(stage_v, out0_hbm.at[pl.ds(c * NPAD + s * PER_SUB, PER_SUB)])
    pltpu.sync_copy(a1_sh.at[sl], stage_v)
    pltpu.sync_copy(stage_v, out1_hbm.at[pl.ds(c * NPAD + s * PER_SUB, PER_SUB)])


_p2_call = pl.kernel(
    _p2_body,
    out_type=(
        jax.ShapeDtypeStruct((CORES * NPAD,), jnp.float32),
        jax.ShapeDtypeStruct((CORES * NPAD,), jnp.float32),
    ),
    mesh=_mesh,
    scratch_types=[
        pltpu.VMEM_SHARED((NPAD,), jnp.float32),
        pltpu.VMEM_SHARED((NPAD,), jnp.float32),
        pltpu.VMEM_SHARED((NPAD,), jnp.float32),
        pltpu.VMEM_SHARED((NPAD,), jnp.float32),
        pltpu.VMEM((U2 * 128,), jnp.int32),
        pltpu.VMEM((U2 * 128,), jnp.int32),
        pltpu.VMEM((U2 * 128,), jnp.float32),
        pltpu.VMEM((U2 * 128,), jnp.float32),
        pltpu.VMEM((PER_SUB,), jnp.float32),
    ],
)


def _prep_body(degp_ref, xp_ref, dinv_ref, v1_ref):
    deg = degp_ref[0] + degp_ref[1] + 1.0
    dinv = lax.rsqrt(deg)
    dinv_ref[...] = dinv
    v1_ref[...] = xp_ref[...] * dinv


_prep_call = pl.pallas_call(
    _prep_body,
    out_shape=(
        jax.ShapeDtypeStruct((NROWS, 128), jnp.float32),
        jax.ShapeDtypeStruct((NROWS, 128), jnp.float32),
    ),
)


def _mid_body(accp_ref, dinv_ref, v1_ref, w1_ref, b1_ref, w2_ref, v2cm_ref):
    dinv = dinv_ref[...]
    s1 = dinv * (accp_ref[0] + accp_ref[1] + v1_ref[...])
    g0 = jnp.zeros_like(s1)
    g1 = jnp.zeros_like(s1)
    for k in range(16):
        h = jnp.maximum(s1 * w1_ref[0, k] + b1_ref[0, k], 0.0)
        g0 += h * w2_ref[k, 0]
        g1 += h * w2_ref[k, 1]
    v2cm_ref[0] = g0 * dinv
    v2cm_ref[1] = g1 * dinv


_mid_call = pl.pallas_call(
    _mid_body,
    in_specs=[
        pl.BlockSpec(memory_space=pltpu.VMEM),
        pl.BlockSpec(memory_space=pltpu.VMEM),
        pl.BlockSpec(memory_space=pltpu.VMEM),
        pl.BlockSpec(memory_space=pltpu.SMEM),
        pl.BlockSpec(memory_space=pltpu.SMEM),
        pl.BlockSpec(memory_space=pltpu.SMEM),
    ],
    out_shape=jax.ShapeDtypeStruct((2, NROWS, 128), jnp.float32),
)


def _final_body(acc2cm_ref, v2cm_ref, dinv_ref, b2_ref, batch_ref, out_ref):
    dinv = dinv_ref[...]
    hs = []
    for ch in range(2):
        s2 = dinv * (acc2cm_ref[0, ch] + acc2cm_ref[1, ch] + v2cm_ref[ch])
        hs.append(jnp.maximum(s2 + b2_ref[0, ch], 0.0))
    bt = batch_ref[...]
    for g in range(NG):
        m = (bt == g).astype(jnp.float32)
        cnt = jnp.maximum(jnp.sum(m), 1.0)
        z0 = jnp.sum(m * hs[0]) / cnt
        z1 = jnp.sum(m * hs[1]) / cnt
        mx = jnp.maximum(z0, z1)
        lse = jnp.log(jnp.exp(z0 - mx) + jnp.exp(z1 - mx)) + mx
        out_ref[g, 0] = z0 - lse
        out_ref[g, 1] = z1 - lse


_final_call = pl.pallas_call(
    _final_body,
    in_specs=[
        pl.BlockSpec(memory_space=pltpu.VMEM),
        pl.BlockSpec(memory_space=pltpu.VMEM),
        pl.BlockSpec(memory_space=pltpu.VMEM),
        pl.BlockSpec(memory_space=pltpu.SMEM),
        pl.BlockSpec(memory_space=pltpu.VMEM),
    ],
    out_specs=pl.BlockSpec(memory_space=pltpu.SMEM),
    out_shape=jax.ShapeDtypeStruct((NG, 2), jnp.float32),
)


def kernel(x, edge_index, batch, W1, b1, W2, b2):
    ei = edge_index.astype(jnp.int32)
    npad_e = EP - NE
    pad_idx = NN + (jnp.arange(npad_e, dtype=jnp.int32) % (NPAD - NN))
    srcp = jnp.concatenate([ei[0], pad_idx])
    dstp = jnp.concatenate([ei[1], pad_idx])
    zerosN = jnp.zeros((NPAD,), jnp.float32)
    ones128 = jnp.ones((U * 128,), jnp.float32)

    degp = _deg_call(dstp, ones128, zerosN)

    xp = jnp.concatenate([x[:, 0], jnp.zeros((NPAD - NN,), jnp.float32)])
    dinv, v1 = _prep_call(degp.reshape(CORES, NROWS, 128), xp.reshape(NROWS, 128))

    acc1p = _p1_call(srcp, dstp, v1.reshape(NPAD), zerosN)

    v2cm = _mid_call(acc1p.reshape(CORES, NROWS, 128), dinv, v1,
                     W1, b1.reshape(1, 16), W2)

    v2flat = v2cm.reshape(2, NPAD)
    acc2c0, acc2c1 = _p2_call(srcp, dstp, v2flat[0], v2flat[1], zerosN)

    acc2cm = jnp.stack(
        [acc2c0.reshape(CORES, NROWS, 128), acc2c1.reshape(CORES, NROWS, 128)],
        axis=1)
    batchp = jnp.concatenate(
        [batch.astype(jnp.int32), jnp.full((NPAD - NN,), NG, jnp.int32)]
    ).reshape(NROWS, 128)

    return _final_call(acc2cm, v2cm, dinv, b2.reshape(1, 2), batchp)
